# baseline (speedup 1.0000x reference)
import functools

import jax
import jax.numpy as jnp
from jax import lax
from jax.experimental import pallas as pl
from jax.experimental.pallas import tpu as pltpu
from jax.experimental.pallas import tpu_sc as plsc

BATCH = 16384
DIM = 32
TBL = 1000000
EPS = 1e-5

_CW = 1024
_NFULL = TBL // _CW
_TAILC = 512
_TAILR = TBL - _NFULL * _CW - _TAILC
_LANES = 16
_RING = 64


@functools.cache
def _make_sc_gather():
  info = plsc.get_sparse_core_info()
  nc, ns = info.num_cores, info.num_subcores
  nw = nc * ns
  max_chunks = (_NFULL + 1 + nw - 1) // nw

  def body(u_idx_hbm, i_idx_hbm, u_t_hbm, i_t_hbm, u_tail_hbm, i_tail_hbm,
           u_out_hbm, i_out_hbm,
           idxbuf, mjl, chunk, stage, sem_c, sem_w):
    wid = lax.axis_index("s") * nc + lax.axis_index("c")
    lane_iota = lax.iota(jnp.int32, _LANES)

    def sweep_table(idx_hbm, t_hbm, tail_hbm, out_hbm):
      pltpu.sync_copy(idx_hbm, idxbuf)

      def filt(g, cnt):
        gb = pl.multiple_of(g * _LANES, _LANES)
        iv = idxbuf[pl.ds(gb, _LANES)]
        cid = lax.shift_right_logical(iv, 10)
        mask = (cid & (nw - 1)) == wid
        jv = gb + lane_iota
        nhit = plsc.all_reduce_population_count(mask)[0]

        @pl.when(nhit > 0)
        def _():
          csum = plsc.cumsum(mask.astype(jnp.int32))
          pos = jnp.where(mask, cnt + csum - 1, BATCH + _LANES - 1)
          plsc.store_scatter(mjl, [pos], jv)

        return cnt + nhit

      cnt = lax.fori_loop(0, BATCH // _LANES, filt, jnp.int32(0))
      ngroups = lax.div(cnt + (_LANES - 1), _LANES)

      def issue_chunk(m, par):
        buf = chunk.at[par]

        @pl.when(m < _NFULL)
        def _():
          pltpu.async_copy(t_hbm.at[:, pl.ds(m * _CW, _CW)], buf, sem_c)

        @pl.when(m == _NFULL)
        def _():
          pltpu.async_copy(
              t_hbm.at[:, pl.ds(_NFULL * _CW, _TAILC)],
              buf.at[:, pl.ds(0, _TAILC)], sem_c)

      def wait_chunk(m, par):
        buf = chunk.at[par]

        @pl.when(m < _NFULL)
        def _():
          pltpu.make_async_copy(
              t_hbm.at[:, pl.ds(0, _CW)], buf, sem_c).wait()

        @pl.when(m == _NFULL)
        def _():
          pltpu.make_async_copy(
              t_hbm.at[:, pl.ds(0, _TAILC)],
              buf.at[:, pl.ds(0, _TAILC)], sem_c).wait()

      def drain_rows(n):
        def drain(q, _):
          pltpu.make_async_copy(
              stage.at[pl.ds(0, 1)],
              out_hbm.at[pl.ds(0, 1)], sem_w).wait()
          return 0

        lax.fori_loop(0, n, drain, 0)

      issue_chunk(wid, 0)

      def do_chunk(t, ordc):
        m = wid + t * nw
        par = t & 1

        @pl.when(m <= _NFULL)
        def _():
          wait_chunk(m, par)
          mn = m + nw

          @pl.when(mn <= _NFULL)
          def _():
            issue_chunk(mn, 1 - par)

        def scan_group(g4, ord4):
          for q in range(4):
            ord4 = scan_one(g4 * 4 + q, ord4)
          return ord4

        def scan_one(g, ordg):
          gb = g * _LANES
          valid = (gb + lane_iota) < cnt
          jv = jnp.where(valid, mjl[pl.ds(gb, _LANES)], 0)
          kv = plsc.load_gather(idxbuf, [jv])
          hit = ((lax.shift_right_logical(kv, 10) == m) & valid)
          hiti = hit.astype(jnp.int32)
          nhit = plsc.all_reduce_population_count(hit)[0]

          wrap = (ordg + nhit) > _RING

          @pl.when(wrap)
          def _():
            drain_rows(ordg)

          base = jnp.where(wrap, 0, ordg)

          @pl.when(nhit > 0)
          def _():
            kk = kv - m * _CW
            slotv = base + plsc.cumsum(hiti) - 1
            cbuf = chunk.at[par]
            for l in range(_LANES):
              @pl.when(hiti[l] == 1)
              def _():
                kkl = kk[l]
                slot = slotv[l]

                @pl.when((m < _NFULL) | (kkl < _TAILC))
                def _():
                  col = jnp.full((_LANES,), kkl, dtype=jnp.int32)
                  g0 = plsc.load_gather(cbuf, [lane_iota, col])
                  g1 = plsc.load_gather(cbuf, [lane_iota + _LANES, col])
                  srow = stage.at[slot]
                  srow[pl.ds(0, _LANES)] = g0
                  srow[pl.ds(_LANES, _LANES)] = g1
                  pltpu.async_copy(
                      stage.at[pl.ds(slot, 1)],
                      out_hbm.at[pl.ds(jv[l], 1)], sem_w)

                @pl.when((m == _NFULL) & (kkl >= _TAILC))
                def _():
                  pltpu.async_copy(
                      tail_hbm.at[pl.ds(kkl - _TAILC, 1)],
                      stage.at[pl.ds(slot, 1)], sem_c).wait()
                  pltpu.async_copy(
                      stage.at[pl.ds(slot, 1)],
                      out_hbm.at[pl.ds(jv[l], 1)], sem_w)

          return base + nhit

        return lax.fori_loop(
            0, jnp.where(m <= _NFULL, (ngroups + 3) >> 2, 0),
            scan_group, ordc)

      ordf = lax.fori_loop(0, max_chunks, do_chunk, jnp.int32(0))
      drain_rows(ordf)

    sweep_table(u_idx_hbm, u_t_hbm, u_tail_hbm, u_out_hbm)
    sweep_table(i_idx_hbm, i_t_hbm, i_tail_hbm, i_out_hbm)

  return pl.kernel(
      body,
      mesh=plsc.VectorSubcoreMesh(core_axis_name="c", subcore_axis_name="s"),
      compiler_params=pltpu.CompilerParams(needs_layout_passes=False),
      out_type=[
          jax.ShapeDtypeStruct((BATCH, DIM), jnp.float32),
          jax.ShapeDtypeStruct((BATCH, DIM), jnp.float32),
      ],
      scratch_types=[
          pltpu.VMEM((BATCH,), jnp.int32),
          pltpu.VMEM((BATCH + 4 * _LANES,), jnp.int32),
          pltpu.VMEM((2, DIM, _CW), jnp.float32),
          pltpu.VMEM((_RING, DIM), jnp.float32),
          pltpu.SemaphoreType.DMA,
          pltpu.SemaphoreType.DMA,
      ],
  )


def _bn(x, gamma, beta):
  mean = jnp.mean(x, axis=0, keepdims=True)
  var = jnp.mean((x - mean) ** 2, axis=0, keepdims=True)
  return (x - mean) * jax.lax.rsqrt(var + EPS) * gamma + beta


def _mlp_body(u_ref, i_ref, w1a_ref, w1b_ref, b1_ref, g1_ref, be1_ref,
              w2_ref, b2_ref, g2_ref, be2_ref, wout_ref, out_ref):
  x = (jnp.dot(u_ref[...], w1a_ref[...], preferred_element_type=jnp.float32)
       + jnp.dot(i_ref[...], w1b_ref[...], preferred_element_type=jnp.float32)
       + b1_ref[...])
  x = jnp.maximum(x, 0.0)
  x = _bn(x, g1_ref[...], be1_ref[...])
  x = jnp.dot(x, w2_ref[...], preferred_element_type=jnp.float32) + b2_ref[...]
  x = jnp.maximum(x, 0.0)
  x = _bn(x, g2_ref[...], be2_ref[...])
  logits = jnp.dot(x, wout_ref[...], preferred_element_type=jnp.float32)
  out_ref[...] = jax.nn.sigmoid(logits)


@jax.jit
def kernel(user_indices, item_indices, user_table, item_table,
           W1, b1, g1, be1, W2, b2, g2, be2, W_out):
  u_emb, i_emb = _make_sc_gather()(
      user_indices, item_indices, user_table.T, item_table.T,
      user_table[TBL - _TAILR:, :], item_table[TBL - _TAILR:, :])

  out = pl.pallas_call(
      _mlp_body,
      out_shape=jax.ShapeDtypeStruct((BATCH, 1), jnp.float32),
  )(u_emb, i_emb, W1[:, :DIM].T, W1[:, DIM:].T,
    b1.reshape(1, -1), g1.reshape(1, -1), be1.reshape(1, -1),
    W2.T, b2.reshape(1, -1), g2.reshape(1, -1), be2.reshape(1, -1),
    W_out.T)
  return out


# speedup vs baseline: 3.6834x; 3.6834x over previous
import functools

import jax
import jax.numpy as jnp
from jax import lax
from jax.experimental import pallas as pl
from jax.experimental.pallas import tpu as pltpu
from jax.experimental.pallas import tpu_sc as plsc

BATCH = 16384
DIM = 32
TBL = 1000000
EPS = 1e-5

_CW = 1024
_NFULL = TBL // _CW
_TAILC = 512
_TAILR = TBL - _NFULL * _CW - _TAILC
_LANES = 16
_RING = 64


@functools.cache
def _make_sc_gather():
  info = plsc.get_sparse_core_info()
  nc, ns = info.num_cores, info.num_subcores
  nw = nc * ns
  max_chunks = (_NFULL + 1 + nw - 1) // nw

  def body(u_idx_hbm, i_idx_hbm, u_t_hbm, i_t_hbm, u_tail_hbm, i_tail_hbm,
           u_out_hbm, i_out_hbm,
           idxbuf, mjl, chunk, stage, sem_c, sem_w):
    wid = lax.axis_index("s") * nc + lax.axis_index("c")
    lane_iota = lax.iota(jnp.int32, _LANES)

    def sweep_table(idx_hbm, t_hbm, tail_hbm, out_hbm):
      pltpu.sync_copy(idx_hbm, idxbuf)

      def filt(g, cnt):
        gb = pl.multiple_of(g * _LANES, _LANES)
        iv = idxbuf[pl.ds(gb, _LANES)]
        cid = lax.shift_right_logical(iv, 10)
        mask = (cid & (nw - 1)) == wid
        jv = gb + lane_iota
        csum = plsc.cumsum(mask.astype(jnp.int32))
        pos = jnp.where(mask, cnt + csum - 1, BATCH + _LANES - 1)
        plsc.store_scatter(mjl, [pos], jv)
        return cnt + csum[_LANES - 1]

      cnt = lax.fori_loop(0, BATCH // _LANES, filt, jnp.int32(0))
      ngroups = lax.div(cnt + (_LANES - 1), _LANES)

      def issue_chunk(m, par):
        buf = chunk.at[par]

        @pl.when(m < _NFULL)
        def _():
          pltpu.async_copy(t_hbm.at[:, pl.ds(m * _CW, _CW)], buf, sem_c)

        @pl.when(m == _NFULL)
        def _():
          pltpu.async_copy(
              t_hbm.at[:, pl.ds(_NFULL * _CW, _TAILC)],
              buf.at[:, pl.ds(0, _TAILC)], sem_c)

      def wait_chunk(m, par):
        buf = chunk.at[par]

        @pl.when(m < _NFULL)
        def _():
          pltpu.make_async_copy(
              t_hbm.at[:, pl.ds(0, _CW)], buf, sem_c).wait()

        @pl.when(m == _NFULL)
        def _():
          pltpu.make_async_copy(
              t_hbm.at[:, pl.ds(0, _TAILC)],
              buf.at[:, pl.ds(0, _TAILC)], sem_c).wait()

      def drain_rows(n):
        def drain(q, _):
          pltpu.make_async_copy(
              stage.at[pl.ds(0, 1)],
              out_hbm.at[pl.ds(0, 1)], sem_w).wait()
          return 0

        lax.fori_loop(0, n, drain, 0)

      issue_chunk(wid, 0)

      def do_chunk(t, ordc):
        m = wid + t * nw
        par = t & 1

        @pl.when(m <= _NFULL)
        def _():
          wait_chunk(m, par)
          mn = m + nw

          @pl.when(mn <= _NFULL)
          def _():
            issue_chunk(mn, 1 - par)

        def scan_group(g, ordg):
          gb = g * _LANES
          valid = (gb + lane_iota) < cnt
          jv = jnp.where(valid, mjl[pl.ds(gb, _LANES)], 0)
          kv = plsc.load_gather(idxbuf, [jv])
          hit = ((lax.shift_right_logical(kv, 10) == m) & valid)
          hiti = hit.astype(jnp.int32)
          csum = plsc.cumsum(hiti)
          nhit = csum[_LANES - 1]

          wrap = (ordg + nhit) > _RING

          @pl.when(wrap)
          def _():
            drain_rows(ordg)

          base = jnp.where(wrap, 0, ordg)

          @pl.when(nhit > 0)
          def _():
            kk = kv - m * _CW
            slotv = base + csum - 1
            cbuf = chunk.at[par]
            for l in range(_LANES):
              @pl.when(hiti[l] == 1)
              def _():
                kkl = kk[l]
                slot = slotv[l]

                @pl.when((m < _NFULL) | (kkl < _TAILC))
                def _():
                  col = jnp.full((_LANES,), kkl, dtype=jnp.int32)
                  g0 = plsc.load_gather(cbuf, [lane_iota, col])
                  g1 = plsc.load_gather(cbuf, [lane_iota + _LANES, col])
                  srow = stage.at[slot]
                  srow[pl.ds(0, _LANES)] = g0
                  srow[pl.ds(_LANES, _LANES)] = g1
                  pltpu.async_copy(
                      stage.at[pl.ds(slot, 1)],
                      out_hbm.at[pl.ds(jv[l], 1)], sem_w)

                @pl.when((m == _NFULL) & (kkl >= _TAILC))
                def _():
                  pltpu.async_copy(
                      tail_hbm.at[pl.ds(kkl - _TAILC, 1)],
                      stage.at[pl.ds(slot, 1)], sem_c).wait()
                  pltpu.async_copy(
                      stage.at[pl.ds(slot, 1)],
                      out_hbm.at[pl.ds(jv[l], 1)], sem_w)

          return base + nhit

        return lax.fori_loop(
            0, jnp.where(m <= _NFULL, ngroups, 0), scan_group, ordc)

      ordf = lax.fori_loop(0, max_chunks, do_chunk, jnp.int32(0))
      drain_rows(ordf)

    sweep_table(u_idx_hbm, u_t_hbm, u_tail_hbm, u_out_hbm)
    sweep_table(i_idx_hbm, i_t_hbm, i_tail_hbm, i_out_hbm)

  return pl.kernel(
      body,
      mesh=plsc.VectorSubcoreMesh(core_axis_name="c", subcore_axis_name="s"),
      compiler_params=pltpu.CompilerParams(needs_layout_passes=False),
      out_type=[
          jax.ShapeDtypeStruct((BATCH, DIM), jnp.float32),
          jax.ShapeDtypeStruct((BATCH, DIM), jnp.float32),
      ],
      scratch_types=[
          pltpu.VMEM((BATCH,), jnp.int32),
          pltpu.VMEM((BATCH + 4 * _LANES,), jnp.int32),
          pltpu.VMEM((2, DIM, _CW), jnp.float32),
          pltpu.VMEM((_RING, DIM), jnp.float32),
          pltpu.SemaphoreType.DMA,
          pltpu.SemaphoreType.DMA,
      ],
  )


def _bn(x, gamma, beta):
  mean = jnp.mean(x, axis=0, keepdims=True)
  var = jnp.mean((x - mean) ** 2, axis=0, keepdims=True)
  return (x - mean) * jax.lax.rsqrt(var + EPS) * gamma + beta


def _mlp_body(u_ref, i_ref, w1a_ref, w1b_ref, b1_ref, g1_ref, be1_ref,
              w2_ref, b2_ref, g2_ref, be2_ref, wout_ref, out_ref):
  x = (jnp.dot(u_ref[...], w1a_ref[...], preferred_element_type=jnp.float32)
       + jnp.dot(i_ref[...], w1b_ref[...], preferred_element_type=jnp.float32)
       + b1_ref[...])
  x = jnp.maximum(x, 0.0)
  x = _bn(x, g1_ref[...], be1_ref[...])
  x = jnp.dot(x, w2_ref[...], preferred_element_type=jnp.float32) + b2_ref[...]
  x = jnp.maximum(x, 0.0)
  x = _bn(x, g2_ref[...], be2_ref[...])
  logits = jnp.dot(x, wout_ref[...], preferred_element_type=jnp.float32)
  out_ref[...] = jax.nn.sigmoid(logits)


@jax.jit
def kernel(user_indices, item_indices, user_table, item_table,
           W1, b1, g1, be1, W2, b2, g2, be2, W_out):
  u_emb, i_emb = _make_sc_gather()(
      user_indices, item_indices, user_table.T, item_table.T,
      user_table[TBL - _TAILR:, :], item_table[TBL - _TAILR:, :])

  out = pl.pallas_call(
      _mlp_body,
      out_shape=jax.ShapeDtypeStruct((BATCH, 1), jnp.float32),
  )(u_emb, i_emb, W1[:, :DIM].T, W1[:, DIM:].T,
    b1.reshape(1, -1), g1.reshape(1, -1), be1.reshape(1, -1),
    W2.T, b2.reshape(1, -1), g2.reshape(1, -1), be2.reshape(1, -1),
    W_out.T)
  return out


# speedup vs baseline: 6.1461x; 1.6686x over previous
import functools

import jax
import jax.numpy as jnp
from jax import lax
from jax.experimental import pallas as pl
from jax.experimental.pallas import tpu as pltpu
from jax.experimental.pallas import tpu_sc as plsc

BATCH = 16384
DIM = 32
TBL = 1000000
EPS = 1e-5

_CW = 1024
_NFULL = TBL // _CW
_TAILC = 512
_TAILR = TBL - _NFULL * _CW - _TAILC
_LANES = 16
_RING = 64


@functools.cache
def _make_sc_gather():
  info = plsc.get_sparse_core_info()
  nc, ns = info.num_cores, info.num_subcores
  nw = nc * ns
  max_chunks = (_NFULL + 1 + nw - 1) // nw

  def body(u_idx_hbm, i_idx_hbm, u_t_hbm, i_t_hbm, u_tail_hbm, i_tail_hbm,
           u_out_hbm, i_out_hbm,
           idxbuf, mjl, chunk, stage, tmp, sem_c, sem_w):
    wid = lax.axis_index("s") * nc + lax.axis_index("c")
    lane_iota = lax.iota(jnp.int32, _LANES)
    trio_off = (lane_iota == 1) * _LANES + (lane_iota == 2) * (2 * _LANES)

    def sweep_table(idx_hbm, t_hbm, tail_hbm, out_hbm):
      pltpu.sync_copy(idx_hbm, idxbuf)

      def filt(g, cnt):
        gb = pl.multiple_of(g * _LANES, _LANES)
        iv = idxbuf[pl.ds(gb, _LANES)]
        cid = lax.shift_right_logical(iv, 10)
        mask = (cid & (nw - 1)) == wid
        jv = gb + lane_iota
        csum = plsc.cumsum(mask.astype(jnp.int32))
        pos = jnp.where(mask, cnt + csum - 1, BATCH + _LANES - 1)
        plsc.store_scatter(mjl, [pos], jv)
        return cnt + csum[_LANES - 1]

      cnt = lax.fori_loop(0, BATCH // _LANES, filt, jnp.int32(0))
      ngroups = lax.div(cnt + (_LANES - 1), _LANES)

      def issue_chunk(m, par):
        buf = chunk.at[par]

        @pl.when(m < _NFULL)
        def _():
          pltpu.async_copy(t_hbm.at[:, pl.ds(m * _CW, _CW)], buf, sem_c)

        @pl.when(m == _NFULL)
        def _():
          pltpu.async_copy(
              t_hbm.at[:, pl.ds(_NFULL * _CW, _TAILC)],
              buf.at[:, pl.ds(0, _TAILC)], sem_c)

      def wait_chunk(m, par):
        buf = chunk.at[par]

        @pl.when(m < _NFULL)
        def _():
          pltpu.make_async_copy(
              t_hbm.at[:, pl.ds(0, _CW)], buf, sem_c).wait()

        @pl.when(m == _NFULL)
        def _():
          pltpu.make_async_copy(
              t_hbm.at[:, pl.ds(0, _TAILC)],
              buf.at[:, pl.ds(0, _TAILC)], sem_c).wait()

      def drain_rows(n):
        def drain(q, _):
          pltpu.make_async_copy(
              stage.at[pl.ds(0, 1)],
              out_hbm.at[pl.ds(0, 1)], sem_w).wait()
          return 0

        lax.fori_loop(0, n, drain, 0)

      issue_chunk(wid, 0)

      def do_chunk(t, ordc):
        m = wid + t * nw
        par = t & 1

        @pl.when(m <= _NFULL)
        def _():
          wait_chunk(m, par)
          mn = m + nw

          @pl.when(mn <= _NFULL)
          def _():
            issue_chunk(mn, 1 - par)

        def scan_group(g, ordg):
          gb = g * _LANES
          valid = (gb + lane_iota) < cnt
          jv = jnp.where(valid, mjl[pl.ds(gb, _LANES)], 0)
          kv = plsc.load_gather(idxbuf, [jv])
          hit = ((lax.shift_right_logical(kv, 10) == m) & valid)
          hiti = hit.astype(jnp.int32)
          csum = plsc.cumsum(hiti)
          nhit = csum[_LANES - 1]

          wrap = (ordg + nhit) > _RING

          @pl.when(wrap)
          def _():
            drain_rows(ordg)

          base = jnp.where(wrap, 0, ordg)

          @pl.when(nhit > 0)
          def _():
            tmp[pl.ds(0, _LANES)] = kv - m * _CW
            tmp[pl.ds(_LANES, _LANES)] = jv
            tmp[pl.ds(2 * _LANES, _LANES)] = base + csum - 1
            cbuf = chunk.at[par]

            def one_hit(h, mrem):
              lane = plsc.all_reduce_ffs(mrem == 1)[0]
              trio = plsc.load_gather(tmp, [lane + trio_off])
              kkl = trio[0]
              jl = trio[1]
              slot = trio[2]

              @pl.when((m < _NFULL) | (kkl < _TAILC))
              def _():
                col = jnp.full((_LANES,), kkl, dtype=jnp.int32)
                g0 = plsc.load_gather(cbuf, [lane_iota, col])
                g1 = plsc.load_gather(cbuf, [lane_iota + _LANES, col])
                srow = stage.at[slot]
                srow[pl.ds(0, _LANES)] = g0
                srow[pl.ds(_LANES, _LANES)] = g1
                pltpu.async_copy(
                    stage.at[pl.ds(slot, 1)],
                    out_hbm.at[pl.ds(jl, 1)], sem_w)

              @pl.when((m == _NFULL) & (kkl >= _TAILC))
              def _():
                pltpu.async_copy(
                    tail_hbm.at[pl.ds(kkl - _TAILC, 1)],
                    stage.at[pl.ds(slot, 1)], sem_c).wait()
                pltpu.async_copy(
                    stage.at[pl.ds(slot, 1)],
                    out_hbm.at[pl.ds(jl, 1)], sem_w)

              return mrem & (lane_iota != lane).astype(jnp.int32)

            lax.fori_loop(0, nhit, one_hit, hiti)

          return base + nhit

        return lax.fori_loop(
            0, jnp.where(m <= _NFULL, ngroups, 0), scan_group, ordc)

      ordf = lax.fori_loop(0, max_chunks, do_chunk, jnp.int32(0))
      drain_rows(ordf)

    sweep_table(u_idx_hbm, u_t_hbm, u_tail_hbm, u_out_hbm)
    sweep_table(i_idx_hbm, i_t_hbm, i_tail_hbm, i_out_hbm)

  return pl.kernel(
      body,
      mesh=plsc.VectorSubcoreMesh(core_axis_name="c", subcore_axis_name="s"),
      compiler_params=pltpu.CompilerParams(needs_layout_passes=False),
      out_type=[
          jax.ShapeDtypeStruct((BATCH, DIM), jnp.float32),
          jax.ShapeDtypeStruct((BATCH, DIM), jnp.float32),
      ],
      scratch_types=[
          pltpu.VMEM((BATCH,), jnp.int32),
          pltpu.VMEM((BATCH + 4 * _LANES,), jnp.int32),
          pltpu.VMEM((2, DIM, _CW), jnp.float32),
          pltpu.VMEM((_RING, DIM), jnp.float32),
          pltpu.VMEM((3 * _LANES,), jnp.int32),
          pltpu.SemaphoreType.DMA,
          pltpu.SemaphoreType.DMA,
      ],
  )


def _bn(x, gamma, beta):
  mean = jnp.mean(x, axis=0, keepdims=True)
  var = jnp.mean((x - mean) ** 2, axis=0, keepdims=True)
  return (x - mean) * jax.lax.rsqrt(var + EPS) * gamma + beta


def _mlp_body(u_ref, i_ref, w1a_ref, w1b_ref, b1_ref, g1_ref, be1_ref,
              w2_ref, b2_ref, g2_ref, be2_ref, wout_ref, out_ref):
  x = (jnp.dot(u_ref[...], w1a_ref[...], preferred_element_type=jnp.float32)
       + jnp.dot(i_ref[...], w1b_ref[...], preferred_element_type=jnp.float32)
       + b1_ref[...])
  x = jnp.maximum(x, 0.0)
  x = _bn(x, g1_ref[...], be1_ref[...])
  x = jnp.dot(x, w2_ref[...], preferred_element_type=jnp.float32) + b2_ref[...]
  x = jnp.maximum(x, 0.0)
  x = _bn(x, g2_ref[...], be2_ref[...])
  logits = jnp.dot(x, wout_ref[...], preferred_element_type=jnp.float32)
  out_ref[...] = jax.nn.sigmoid(logits)


@jax.jit
def kernel(user_indices, item_indices, user_table, item_table,
           W1, b1, g1, be1, W2, b2, g2, be2, W_out):
  u_emb, i_emb = _make_sc_gather()(
      user_indices, item_indices, user_table.T, item_table.T,
      user_table[TBL - _TAILR:, :], item_table[TBL - _TAILR:, :])

  out = pl.pallas_call(
      _mlp_body,
      out_shape=jax.ShapeDtypeStruct((BATCH, 1), jnp.float32),
  )(u_emb, i_emb, W1[:, :DIM].T, W1[:, DIM:].T,
    b1.reshape(1, -1), g1.reshape(1, -1), be1.reshape(1, -1),
    W2.T, b2.reshape(1, -1), g2.reshape(1, -1), be2.reshape(1, -1),
    W_out.T)
  return out


# speedup vs baseline: 6.2130x; 1.0109x over previous
import functools

import jax
import jax.numpy as jnp
from jax import lax
from jax.experimental import pallas as pl
from jax.experimental.pallas import tpu as pltpu
from jax.experimental.pallas import tpu_sc as plsc

BATCH = 16384
DIM = 32
TBL = 1000000
EPS = 1e-5

_CW = 1024
_NFULL = TBL // _CW
_TAILC = 512
_TAILR = TBL - _NFULL * _CW - _TAILC
_LANES = 16
_RING = 64


@functools.cache
def _make_sc_gather():
  info = plsc.get_sparse_core_info()
  nc, ns = info.num_cores, info.num_subcores
  nw = nc * ns
  max_chunks = (_NFULL + 1 + nw - 1) // nw

  def body(u_idx_hbm, i_idx_hbm, u_t_hbm, i_t_hbm, u_tail_hbm, i_tail_hbm,
           u_out_hbm, i_out_hbm,
           idxbuf, mjl, chunk, stage, tmp, sem_c, sem_w):
    wid = lax.axis_index("s") * nc + lax.axis_index("c")
    lane_iota = lax.iota(jnp.int32, _LANES)
    trio_off = (lane_iota == 1) * _LANES + (lane_iota == 2) * (2 * _LANES)

    def sweep_table(idx_hbm, t_hbm, tail_hbm, out_hbm):
      pltpu.sync_copy(idx_hbm, idxbuf)
      pltpu.async_copy(
          t_hbm.at[:, pl.ds(wid * _CW, _CW)], chunk.at[0], sem_c)

      def filt_one(g, cnt):
        gb = pl.multiple_of(g * _LANES, _LANES)
        iv = idxbuf[pl.ds(gb, _LANES)]
        cid = lax.shift_right_logical(iv, 10)
        mask = (cid & (nw - 1)) == wid
        jv = gb + lane_iota
        csum = plsc.cumsum(mask.astype(jnp.int32))
        pos = jnp.where(mask, cnt + csum - 1, BATCH + _LANES - 1)
        plsc.store_scatter(mjl, [pos], jv)
        return cnt + csum[_LANES - 1]

      def filt(p, cnt):
        cnt = filt_one(p * 2, cnt)
        return filt_one(p * 2 + 1, cnt)

      cnt = lax.fori_loop(0, BATCH // (2 * _LANES), filt, jnp.int32(0))
      ngroups = lax.div(cnt + (_LANES - 1), _LANES)

      def issue_chunk(m, par):
        buf = chunk.at[par]

        @pl.when(m < _NFULL)
        def _():
          pltpu.async_copy(t_hbm.at[:, pl.ds(m * _CW, _CW)], buf, sem_c)

        @pl.when(m == _NFULL)
        def _():
          pltpu.async_copy(
              t_hbm.at[:, pl.ds(_NFULL * _CW, _TAILC)],
              buf.at[:, pl.ds(0, _TAILC)], sem_c)

      def wait_chunk(m, par):
        buf = chunk.at[par]

        @pl.when(m < _NFULL)
        def _():
          pltpu.make_async_copy(
              t_hbm.at[:, pl.ds(0, _CW)], buf, sem_c).wait()

        @pl.when(m == _NFULL)
        def _():
          pltpu.make_async_copy(
              t_hbm.at[:, pl.ds(0, _TAILC)],
              buf.at[:, pl.ds(0, _TAILC)], sem_c).wait()

      def drain_rows(n):
        def drain(q, _):
          pltpu.make_async_copy(
              stage.at[pl.ds(0, 1)],
              out_hbm.at[pl.ds(0, 1)], sem_w).wait()
          return 0

        lax.fori_loop(0, n, drain, 0)

      def do_chunk(t, ordc):
        m = wid + t * nw
        par = t & 1

        @pl.when(m <= _NFULL)
        def _():
          wait_chunk(m, par)
          mn = m + nw

          @pl.when(mn <= _NFULL)
          def _():
            issue_chunk(mn, 1 - par)

        def scan_group(g, ordg):
          gb = g * _LANES
          valid = (gb + lane_iota) < cnt
          jv = jnp.where(valid, mjl[pl.ds(gb, _LANES)], 0)
          kv = plsc.load_gather(idxbuf, [jv])
          hit = ((lax.shift_right_logical(kv, 10) == m) & valid)
          hiti = hit.astype(jnp.int32)
          csum = plsc.cumsum(hiti)
          nhit = csum[_LANES - 1]

          wrap = (ordg + nhit) > _RING

          @pl.when(wrap)
          def _():
            drain_rows(ordg)

          base = jnp.where(wrap, 0, ordg)

          @pl.when(nhit > 0)
          def _():
            tmp[pl.ds(0, _LANES)] = kv - m * _CW
            tmp[pl.ds(_LANES, _LANES)] = jv
            tmp[pl.ds(2 * _LANES, _LANES)] = base + csum - 1
            cbuf = chunk.at[par]

            def one_hit(h, mrem):
              lane = plsc.all_reduce_ffs(mrem == 1)[0]
              trio = plsc.load_gather(tmp, [lane + trio_off])
              kkl = trio[0]
              jl = trio[1]
              slot = trio[2]

              @pl.when((m < _NFULL) | (kkl < _TAILC))
              def _():
                col = jnp.full((_LANES,), kkl, dtype=jnp.int32)
                g0 = plsc.load_gather(cbuf, [lane_iota, col])
                g1 = plsc.load_gather(cbuf, [lane_iota + _LANES, col])
                srow = stage.at[slot]
                srow[pl.ds(0, _LANES)] = g0
                srow[pl.ds(_LANES, _LANES)] = g1
                pltpu.async_copy(
                    stage.at[pl.ds(slot, 1)],
                    out_hbm.at[pl.ds(jl, 1)], sem_w)

              @pl.when((m == _NFULL) & (kkl >= _TAILC))
              def _():
                pltpu.async_copy(
                    tail_hbm.at[pl.ds(kkl - _TAILC, 1)],
                    stage.at[pl.ds(slot, 1)], sem_c).wait()
                pltpu.async_copy(
                    stage.at[pl.ds(slot, 1)],
                    out_hbm.at[pl.ds(jl, 1)], sem_w)

              return mrem & (lane_iota != lane).astype(jnp.int32)

            lax.fori_loop(0, nhit, one_hit, hiti)

          return base + nhit

        return lax.fori_loop(
            0, jnp.where(m <= _NFULL, ngroups, 0), scan_group, ordc)

      ordf = lax.fori_loop(0, max_chunks, do_chunk, jnp.int32(0))
      drain_rows(ordf)

    sweep_table(u_idx_hbm, u_t_hbm, u_tail_hbm, u_out_hbm)
    sweep_table(i_idx_hbm, i_t_hbm, i_tail_hbm, i_out_hbm)

  return pl.kernel(
      body,
      mesh=plsc.VectorSubcoreMesh(core_axis_name="c", subcore_axis_name="s"),
      compiler_params=pltpu.CompilerParams(needs_layout_passes=False),
      out_type=[
          jax.ShapeDtypeStruct((BATCH, DIM), jnp.float32),
          jax.ShapeDtypeStruct((BATCH, DIM), jnp.float32),
      ],
      scratch_types=[
          pltpu.VMEM((BATCH,), jnp.int32),
          pltpu.VMEM((BATCH + 4 * _LANES,), jnp.int32),
          pltpu.VMEM((2, DIM, _CW), jnp.float32),
          pltpu.VMEM((_RING, DIM), jnp.float32),
          pltpu.VMEM((3 * _LANES,), jnp.int32),
          pltpu.SemaphoreType.DMA,
          pltpu.SemaphoreType.DMA,
      ],
  )


def _bn(x, gamma, beta):
  mean = jnp.mean(x, axis=0, keepdims=True)
  var = jnp.mean((x - mean) ** 2, axis=0, keepdims=True)
  return (x - mean) * jax.lax.rsqrt(var + EPS) * gamma + beta


def _mlp_body(u_ref, i_ref, w1a_ref, w1b_ref, b1_ref, g1_ref, be1_ref,
              w2_ref, b2_ref, g2_ref, be2_ref, wout_ref, out_ref):
  x = (jnp.dot(u_ref[...], w1a_ref[...], preferred_element_type=jnp.float32)
       + jnp.dot(i_ref[...], w1b_ref[...], preferred_element_type=jnp.float32)
       + b1_ref[...])
  x = jnp.maximum(x, 0.0)
  x = _bn(x, g1_ref[...], be1_ref[...])
  x = jnp.dot(x, w2_ref[...], preferred_element_type=jnp.float32) + b2_ref[...]
  x = jnp.maximum(x, 0.0)
  x = _bn(x, g2_ref[...], be2_ref[...])
  logits = jnp.dot(x, wout_ref[...], preferred_element_type=jnp.float32)
  out_ref[...] = jax.nn.sigmoid(logits)


@jax.jit
def kernel(user_indices, item_indices, user_table, item_table,
           W1, b1, g1, be1, W2, b2, g2, be2, W_out):
  u_emb, i_emb = _make_sc_gather()(
      user_indices, item_indices, user_table.T, item_table.T,
      user_table[TBL - _TAILR:, :], item_table[TBL - _TAILR:, :])

  out = pl.pallas_call(
      _mlp_body,
      out_shape=jax.ShapeDtypeStruct((BATCH, 1), jnp.float32),
  )(u_emb, i_emb, W1[:, :DIM].T, W1[:, DIM:].T,
    b1.reshape(1, -1), g1.reshape(1, -1), be1.reshape(1, -1),
    W2.T, b2.reshape(1, -1), g2.reshape(1, -1), be2.reshape(1, -1),
    W_out.T)
  return out
